# baseline (speedup 1.0000x reference)
import numpy as np
import jax
import jax.numpy as jnp
from jax import lax
from jax.experimental import pallas as pl
from jax.experimental.pallas import tpu as pltpu
from jax.experimental.pallas import tpu_sc as plsc

HIDDEN = 768
SD = 64
MAXP = 24
G = 24
P = G * G
TBL = 2 * MAXP - 1
TBL2 = TBL * TBL

NC, NS = 2, 16
NW_USED = 24
ROWS_W = P // NW_USED
LANES = 16
VECS_ROW = P // LANES

_Q = np.arange(P)
_D_NP = (TBL * (_Q // G) + (_Q % G)).astype(np.int32)


def _add_body(row_ref, col_ref, w_ref, b_ref, x_ref, o_ref, pe_ref):
    @pl.when(pl.program_id(0) == 0)
    def _():
        r_proj = jnp.dot(row_ref[...], w_ref[: SD // 2, :],
                         preferred_element_type=jnp.float32)
        c_proj = jnp.dot(col_ref[...], w_ref[SD // 2:, :],
                         preferred_element_type=jnp.float32)
        c_plus_b = c_proj + b_ref[...][None, :]
        for r in range(G):
            pe_ref[r * G:(r + 1) * G, :] = c_plus_b + r_proj[r:r + 1, :]
    o_ref[...] = x_ref[...] + pe_ref[...][None]


def _pos_add(x, row_embed, col_embed, proj_w, proj_b, bb=8):
    b = x.shape[0]
    const = lambda i: (0, 0)
    return pl.pallas_call(
        _add_body,
        grid=(b // bb,),
        in_specs=[
            pl.BlockSpec((MAXP, SD // 2), const),
            pl.BlockSpec((MAXP, SD // 2), const),
            pl.BlockSpec((SD, HIDDEN), const),
            pl.BlockSpec((HIDDEN,), lambda i: (0,)),
            pl.BlockSpec((bb, P, HIDDEN), lambda i: (i, 0, 0)),
        ],
        out_specs=pl.BlockSpec((bb, P, HIDDEN), lambda i: (i, 0, 0)),
        out_shape=jax.ShapeDtypeStruct((b, P, HIDDEN), jnp.float32),
        scratch_shapes=[pltpu.VMEM((P, HIDDEN), jnp.float32)],
    )(row_embed, col_embed, proj_w, proj_b, x)


def _bias_body(tbl_hbm, out_hbm, tbl_v, out_v):
    wid = lax.axis_index("s") * NC + lax.axis_index("c")

    @pl.when(wid < NW_USED)
    def _():
        pltpu.sync_copy(tbl_hbm, tbl_v)
        r1 = wid + (MAXP - 1)

        for i in range(ROWS_W):
            c1 = i + (MAXP - 1)

            def body(j, carry, c1=c1, i=i):
                r2, c2 = carry
                off = j * LANES
                vals = plsc.load_gather(tbl_v, [r1 - r2, c1 - c2])
                out_v[i, pl.ds(off, LANES)] = vals
                c2n = c2 + LANES
                wrap = c2n >= G
                return (r2 + wrap.astype(jnp.int32),
                        jnp.where(wrap, c2n - G, c2n))

            lax.fori_loop(
                0, VECS_ROW, body,
                (jnp.zeros((LANES,), jnp.int32), lax.iota(jnp.int32, LANES)),
                unroll=4)
        pltpu.sync_copy(out_v, out_hbm.at[pl.ds(wid * ROWS_W, ROWS_W)])


def _bias_gather(tbl):
    mesh = plsc.VectorSubcoreMesh(
        core_axis_name="c", subcore_axis_name="s",
        num_cores=NC, num_subcores=NS)
    k = pl.kernel(
        _bias_body,
        out_type=jax.ShapeDtypeStruct((P, P), jnp.float32),
        mesh=mesh,
        compiler_params=pltpu.CompilerParams(needs_layout_passes=False),
        scratch_types=[
            pltpu.VMEM((TBL + 1, TBL + 1), jnp.float32),
            pltpu.VMEM((ROWS_W, P), jnp.float32),
        ],
    )
    return k(tbl)


def kernel(x, row_embed, col_embed, proj_w, proj_b, rel_bias, return_bias):
    gate = (jnp.asarray(return_bias) != 0).astype(jnp.float32)
    tbl = jnp.pad(rel_bias * gate, ((0, 1), (0, 1)))
    bias = _bias_gather(tbl)
    out = _pos_add(x, row_embed, col_embed, proj_w, proj_b, bb=8)
    return (out, bias)


# speedup vs baseline: 1.0358x; 1.0358x over previous
import numpy as np
import jax
import jax.numpy as jnp
from jax import lax
from jax.experimental import pallas as pl
from jax.experimental.pallas import tpu as pltpu
from jax.experimental.pallas import tpu_sc as plsc

HIDDEN = 768
SD = 64
MAXP = 24
G = 24
P = G * G
TBL = 2 * MAXP - 1
TBL2 = TBL * TBL

NC, NS = 2, 16
NW_USED = 12
ROWS_W = P // NW_USED
LANES = 16
VECS_ROW = P // LANES

_Q = np.arange(P)
_D_NP = (TBL * (_Q // G) + (_Q % G)).astype(np.int32)


def _add_body(row_ref, col_ref, w_ref, b_ref, x_ref, o_ref, pe_ref):
    @pl.when(pl.program_id(0) == 0)
    def _():
        r_proj = jnp.dot(row_ref[...], w_ref[: SD // 2, :],
                         preferred_element_type=jnp.float32)
        c_proj = jnp.dot(col_ref[...], w_ref[SD // 2:, :],
                         preferred_element_type=jnp.float32)
        c_plus_b = c_proj + b_ref[...][None, :]
        for r in range(G):
            pe_ref[r * G:(r + 1) * G, :] = c_plus_b + r_proj[r:r + 1, :]
    o_ref[...] = x_ref[...] + pe_ref[...][None]


def _pos_add(x, row_embed, col_embed, proj_w, proj_b, bb=8):
    b = x.shape[0]
    const = lambda i: (0, 0)
    return pl.pallas_call(
        _add_body,
        grid=(b // bb,),
        in_specs=[
            pl.BlockSpec((MAXP, SD // 2), const),
            pl.BlockSpec((MAXP, SD // 2), const),
            pl.BlockSpec((SD, HIDDEN), const),
            pl.BlockSpec((HIDDEN,), lambda i: (0,)),
            pl.BlockSpec((bb, P, HIDDEN), lambda i: (i, 0, 0)),
        ],
        out_specs=pl.BlockSpec((bb, P, HIDDEN), lambda i: (i, 0, 0)),
        out_shape=jax.ShapeDtypeStruct((b, P, HIDDEN), jnp.float32),
        scratch_shapes=[pltpu.VMEM((P, HIDDEN), jnp.float32)],
    )(row_embed, col_embed, proj_w, proj_b, x)


def _bias_body(tbl_hbm, out_hbm, tbl_v, out_v):
    wid = lax.axis_index("s")

    @pl.when(wid < NW_USED)
    def _():
        pltpu.sync_copy(tbl_hbm, tbl_v)
        g_per_w = ROWS_W // G

        for i in range(ROWS_W):
            r1 = g_per_w * wid + (i // G) + (MAXP - 1)
            c1 = (i % G) + (MAXP - 1)

            def body(j, carry, r1=r1, c1=c1, i=i):
                r2, c2 = carry
                off = j * LANES
                vals = plsc.load_gather(tbl_v, [r1 - r2, c1 - c2])
                out_v[i, pl.ds(off, LANES)] = vals
                c2n = c2 + LANES
                wrap = c2n >= G
                return (r2 + wrap.astype(jnp.int32),
                        jnp.where(wrap, c2n - G, c2n))

            lax.fori_loop(
                0, VECS_ROW, body,
                (jnp.zeros((LANES,), jnp.int32), lax.iota(jnp.int32, LANES)),
                unroll=4)
        pltpu.sync_copy(out_v, out_hbm.at[pl.ds(wid * ROWS_W, ROWS_W)])


def _bias_gather(tbl):
    mesh = plsc.VectorSubcoreMesh(
        core_axis_name="c", subcore_axis_name="s",
        num_cores=1, num_subcores=NS)
    k = pl.kernel(
        _bias_body,
        out_type=jax.ShapeDtypeStruct((P, P), jnp.float32),
        mesh=mesh,
        compiler_params=pltpu.CompilerParams(needs_layout_passes=False),
        scratch_types=[
            pltpu.VMEM((TBL + 1, TBL + 1), jnp.float32),
            pltpu.VMEM((ROWS_W, P), jnp.float32),
        ],
    )
    return k(tbl)


def kernel(x, row_embed, col_embed, proj_w, proj_b, rel_bias, return_bias):
    gate = (jnp.asarray(return_bias) != 0).astype(jnp.float32)
    tbl = jnp.pad(rel_bias * gate, ((0, 1), (0, 1)))
    bias = _bias_gather(tbl)
    out = _pos_add(x, row_embed, col_embed, proj_w, proj_b, bb=8)
    return (out, bias)
